# Initial kernel scaffold; baseline (speedup 1.0000x reference)
import jax
import jax.numpy as jnp
from jax.experimental import pallas as pl


def kernel(x, pos, edge_index, W_self, b_self, W_local, b_local, W_global, b_global):
    raise NotImplementedError("write your pallas kernel here")


# speedup vs baseline: 15.4554x; 15.4554x over previous
import functools

import jax
import jax.numpy as jnp
from jax import lax
from jax.experimental import pallas as pl
from jax.experimental.pallas import tpu as pltpu
from jax.experimental.pallas import tpu_sc as plsc

N = 10000
NPAD = 10112
E = 320000
CH = 128
NWORK = 32
NITER = 80
EPAD = NWORK * NITER * CH
WEXT = 144
RZ = NPAD // 16
BLK = 1000


def _sc_segment_sum(xext, src, dst, zeros):
    mesh = plsc.VectorSubcoreMesh(core_axis_name="c", subcore_axis_name="s")

    @functools.partial(
        pl.kernel,
        mesh=mesh,
        compiler_params=pltpu.CompilerParams(use_tc_tiling_on_sc=False),
        out_type=jax.ShapeDtypeStruct((2 * N, WEXT), jnp.float32),
        scratch_types=[
            pltpu.VMEM((4, CH), jnp.int32),
            pltpu.VMEM((4, CH), jnp.int32),
            pltpu.VMEM((2, CH, WEXT), jnp.float32),
            pltpu.VMEM_SHARED((NPAD, WEXT), jnp.float32),
            pltpu.SemaphoreType.DMA((2,)),
            pltpu.SemaphoreType.DMA((2,)),
            pltpu.SemaphoreType.DMA((4,)),
            pltpu.SemaphoreType.DMA((4,)),
        ],
    )
    def body(xext_hbm, src_hbm, dst_hbm, zeros_hbm, out_hbm, sidx, didx, rows, acc,
             semg, semsc, semis, semid):
        cid = lax.axis_index("c")
        sid = lax.axis_index("s")
        wid = sid * 2 + cid
        base0 = wid * NITER * CH

        def idx_start(j, js):
            pltpu.async_copy(src_hbm.at[pl.ds(base0 + j * CH, CH)],
                             sidx.at[js], semis.at[js])
            pltpu.async_copy(dst_hbm.at[pl.ds(base0 + j * CH, CH)],
                             didx.at[js], semid.at[js])

        def idx_wait(js):
            pltpu.make_async_copy(src_hbm.at[pl.ds(0, CH)],
                                  sidx.at[js], semis.at[js]).wait()
            pltpu.make_async_copy(dst_hbm.at[pl.ds(0, CH)],
                                  didx.at[js], semid.at[js]).wait()

        def gather_start(js, s):
            pltpu.async_copy(xext_hbm.at[sidx.at[js]], rows.at[s], semg.at[s])

        def gather_wait(js, s):
            pltpu.make_async_copy(xext_hbm.at[sidx.at[js]], rows.at[s],
                                  semg.at[s]).wait()

        def scatter_start(js, s):
            pltpu.async_copy(rows.at[s], acc.at[didx.at[js]], semsc.at[s], add=True)

        def scatter_wait(js, s):
            pltpu.make_async_copy(rows.at[s], acc.at[didx.at[js]],
                                  semsc.at[s]).wait()

        for t in range(4):
            idx_start(t, t)
        idx_wait(0)
        gather_start(0, 0)
        pltpu.sync_copy(zeros_hbm, acc.at[pl.ds(sid * RZ, RZ)])
        plsc.subcore_barrier()

        def pipe(jj, carry):
            j0 = jj * 4
            for t in range(4):
                j = j0 + t
                s = t % 2
                gather_wait(t, s)
                scatter_start(t, s)

                if t == 0:
                    @pl.when(jj > 0)
                    def _():
                        scatter_wait(3, 1)
                else:
                    scatter_wait(t - 1, 1 - s)

                @pl.when((j > 0) & (j + 3 < NITER))
                def _():
                    idx_start(j + 3, (t + 3) % 4)

                @pl.when(j + 1 < NITER)
                def _():
                    idx_wait((t + 1) % 4)
                    gather_start((t + 1) % 4, 1 - s)
            return carry

        lax.fori_loop(0, NITER // 4, pipe, 0)
        scatter_wait(3, 1)
        plsc.subcore_barrier()

        @pl.when(sid < 10)
        def _():
            pltpu.sync_copy(acc.at[pl.ds(sid * 1000, 1000)],
                            out_hbm.at[pl.ds(cid * N + sid * 1000, 1000)])

    return body(xext, src, dst, zeros)


def _tc_combine(part, xext, wl, ws, wg, bl, bs, bg):
    nb = N // BLK

    def dot_t(a, w):
        return lax.dot_general(a, w, (((1,), (1,)), ((), ())),
                               preferred_element_type=jnp.float32)

    def body(p0, p1, xer, wlr, wsr, wgr, blr, bsr, bgr, outr):
        pa = p0[...] + p1[...]
        deg = pa[:, 131:132]
        xe = xer[...]
        xv = xe[:, :128]
        wlv = wlr[...]
        wgv = wgr[...]
        t1 = dot_t(pa[:, :131], wlv) + deg * blr[...]
        t3 = deg * dot_t(xe[:, 128:131], wlv[:, 128:131])
        t2 = dot_t(xe[:, :131], wsr[...]) + bsr[...]
        out1 = t1 - t3 + t2
        out2 = dot_t(out1, wgv[:, :128]) + dot_t(xv, wgv[:, 128:]) + bgr[...]
        outr[...] = xv + out2

    def full(shape):
        return pl.BlockSpec(shape, lambda i: (0, 0))

    return pl.pallas_call(
        body,
        grid=(nb,),
        in_specs=[
            pl.BlockSpec((BLK, WEXT), lambda i: (i, 0)),
            pl.BlockSpec((BLK, WEXT), lambda i: (i + nb, 0)),
            pl.BlockSpec((BLK, WEXT), lambda i: (i, 0)),
            full((128, 131)), full((128, 131)), full((128, 256)),
            full((1, 128)), full((1, 128)), full((1, 128)),
        ],
        out_specs=pl.BlockSpec((BLK, 128), lambda i: (i, 0)),
        out_shape=jax.ShapeDtypeStruct((N, 128), jnp.float32),
    )(part, part, xext, wl, ws, wg, bl, bs, bg)


def kernel(x, pos, edge_index, W_self, b_self, W_local, b_local, W_global, b_global):
    f32 = jnp.float32
    i32 = jnp.int32
    src = jnp.concatenate(
        [edge_index[0].astype(i32), jnp.arange(EPAD - E, dtype=i32) % N])
    dst = jnp.concatenate(
        [edge_index[1].astype(i32),
         N + jnp.arange(EPAD - E, dtype=i32) % (NPAD - N)])
    xext = jnp.concatenate(
        [x, pos, jnp.ones((N, 1), f32), jnp.zeros((N, WEXT - 132), f32)], axis=1)
    zeros = jnp.zeros((RZ, WEXT), f32)
    part = _sc_segment_sum(xext, src, dst, zeros)
    return _tc_combine(part, xext, W_local, W_self, W_global,
                       b_local[None, :], b_self[None, :], b_global[None, :])
